# baseline (speedup 1.0000x reference)
import jax
import jax.numpy as jnp
from jax.experimental import pallas as pl
from jax.experimental.pallas import tpu as pltpu

BATCH = 16384
FIELDS = 4
EMBED = 64
ROWS = FIELDS * EMBED
CBLK = 8192
NBLK = BATCH // CBLK


def _tc_pipe_body(ft_ref, bias_ref, xt_hbm, out_ref, buf0, buf1, sem0, sem1):
    bufs = (buf0, buf1)
    sems = (sem0, sem1)

    def start(j):
        p = j % 2
        c = pltpu.make_async_copy(
            xt_hbm.at[:, pl.ds(j * CBLK, CBLK)], bufs[p], sems[p])
        c.start()
        return c

    pend = start(0)
    for j in range(NBLK):
        p = j % 2
        nxt = start(j + 1) if j + 1 < NBLK else None
        pend.wait()
        x = bufs[p][...]
        v0 = x[0:64, :]
        v1 = x[64:128, :]
        v2 = x[128:192, :]
        v3 = x[192:256, :]
        t = v0 * v1 + v2 * v3 + (v0 + v1) * (v2 + v3)
        inter = jnp.sum(t, axis=0, keepdims=True)
        ft = jnp.sum(ft_ref[:, pl.ds(j * CBLK, CBLK)], axis=0, keepdims=True)
        out_ref[:, pl.ds(j * CBLK, CBLK)] = bias_ref[0, 0] + ft + inter
        pend = nxt


def kernel(first_embeddings, second_embeddings, bias):
    xt = jnp.transpose(second_embeddings, (1, 2, 0)).reshape(ROWS, BATCH)
    ft = jnp.transpose(first_embeddings, (1, 0))
    xt = pltpu.with_memory_space_constraint(xt, pltpu.MemorySpace.HBM)
    out = pl.pallas_call(
        _tc_pipe_body,
        in_specs=[
            pl.BlockSpec((FIELDS, BATCH), lambda: (0, 0)),
            pl.BlockSpec((1, 1), lambda: (0, 0)),
            pl.BlockSpec(memory_space=pl.ANY),
        ],
        out_specs=pl.BlockSpec((1, BATCH), lambda: (0, 0)),
        out_shape=jax.ShapeDtypeStruct((1, BATCH), jnp.float32),
        scratch_shapes=[
            pltpu.VMEM((ROWS, CBLK), jnp.float32),
            pltpu.VMEM((ROWS, CBLK), jnp.float32),
            pltpu.SemaphoreType.DMA,
            pltpu.SemaphoreType.DMA,
        ],
    )(ft, bias.reshape(1, 1), xt)
    return out.reshape(BATCH)


# speedup vs baseline: 1.0940x; 1.0940x over previous
import jax
import jax.numpy as jnp
from jax.experimental import pallas as pl
from jax.experimental.pallas import tpu as pltpu

BATCH = 16384
FIELDS = 4
EMBED = 64
ROWS = FIELDS * EMBED
CBLK = 4096
NBLK = BATCH // CBLK


def _tc_pipe_body(ft_ref, bias_ref, xt_hbm, out_ref,
                  buf0, buf1, buf2, buf3, sem0, sem1, sem2, sem3):
    bufs = (buf0, buf1, buf2, buf3)
    sems = (sem0, sem1, sem2, sem3)

    copies = []
    for j in range(NBLK):
        c = pltpu.make_async_copy(
            xt_hbm.at[:, pl.ds(j * CBLK, CBLK)], bufs[j], sems[j])
        c.start()
        copies.append(c)
    for j in range(NBLK):
        copies[j].wait()
        x = bufs[j][...]
        v0 = x[0:64, :]
        v1 = x[64:128, :]
        v2 = x[128:192, :]
        v3 = x[192:256, :]
        t = v0 * v1 + v2 * v3 + (v0 + v1) * (v2 + v3)
        inter = jnp.sum(t, axis=0, keepdims=True)
        ft = jnp.sum(ft_ref[:, pl.ds(j * CBLK, CBLK)], axis=0, keepdims=True)
        out_ref[:, pl.ds(j * CBLK, CBLK)] = bias_ref[0, 0] + ft + inter


def kernel(first_embeddings, second_embeddings, bias):
    xt = jnp.transpose(second_embeddings, (1, 2, 0)).reshape(ROWS, BATCH)
    ft = jnp.transpose(first_embeddings, (1, 0))
    xt = pltpu.with_memory_space_constraint(xt, pltpu.MemorySpace.HBM)
    out = pl.pallas_call(
        _tc_pipe_body,
        in_specs=[
            pl.BlockSpec((FIELDS, BATCH), lambda: (0, 0)),
            pl.BlockSpec((1, 1), lambda: (0, 0)),
            pl.BlockSpec(memory_space=pl.ANY),
        ],
        out_specs=pl.BlockSpec((1, BATCH), lambda: (0, 0)),
        out_shape=jax.ShapeDtypeStruct((1, BATCH), jnp.float32),
        scratch_shapes=(
            [pltpu.VMEM((ROWS, CBLK), jnp.float32) for _ in range(NBLK)]
            + [pltpu.SemaphoreType.DMA for _ in range(NBLK)]
        ),
    )(ft, bias.reshape(1, 1), xt)
    return out.reshape(BATCH)


# speedup vs baseline: 1.1027x; 1.0080x over previous
import jax
import jax.numpy as jnp
from jax.experimental import pallas as pl
from jax.experimental.pallas import tpu as pltpu

BATCH = 16384
FIELDS = 4
EMBED = 64
ROWS = FIELDS * EMBED
CBLK = 2048
NBLK = BATCH // CBLK


def _tc_pipe_body(ft_ref, bias_ref, xt_hbm, out_ref, *scratch):
    bufs = scratch[:NBLK]
    sems = scratch[NBLK:]

    copies = []
    for j in range(NBLK):
        c = pltpu.make_async_copy(
            xt_hbm.at[:, pl.ds(j * CBLK, CBLK)], bufs[j], sems[j])
        c.start()
        copies.append(c)
    for j in range(NBLK):
        copies[j].wait()
        x = bufs[j][...]
        v0 = x[0:64, :]
        v1 = x[64:128, :]
        v2 = x[128:192, :]
        v3 = x[192:256, :]
        t = v0 * v1 + v2 * v3 + (v0 + v1) * (v2 + v3)
        inter = jnp.sum(t, axis=0, keepdims=True)
        ft = jnp.sum(ft_ref[:, pl.ds(j * CBLK, CBLK)], axis=0, keepdims=True)
        out_ref[:, pl.ds(j * CBLK, CBLK)] = bias_ref[0, 0] + ft + inter


def kernel(first_embeddings, second_embeddings, bias):
    xt = jnp.transpose(second_embeddings, (1, 2, 0)).reshape(ROWS, BATCH)
    ft = jnp.transpose(first_embeddings, (1, 0))
    xt = pltpu.with_memory_space_constraint(xt, pltpu.MemorySpace.HBM)
    out = pl.pallas_call(
        _tc_pipe_body,
        in_specs=[
            pl.BlockSpec((FIELDS, BATCH), lambda: (0, 0)),
            pl.BlockSpec((1, 1), lambda: (0, 0)),
            pl.BlockSpec(memory_space=pl.ANY),
        ],
        out_specs=pl.BlockSpec((1, BATCH), lambda: (0, 0)),
        out_shape=jax.ShapeDtypeStruct((1, BATCH), jnp.float32),
        scratch_shapes=(
            [pltpu.VMEM((ROWS, CBLK), jnp.float32) for _ in range(NBLK)]
            + [pltpu.SemaphoreType.DMA for _ in range(NBLK)]
        ),
    )(ft, bias.reshape(1, 1), xt)
    return out.reshape(BATCH)


# speedup vs baseline: 1.1416x; 1.0353x over previous
import jax
import jax.numpy as jnp
from jax.experimental import pallas as pl
from jax.experimental.pallas import tpu as pltpu

BATCH = 16384
FIELDS = 4
EMBED = 64
ROWS = FIELDS * EMBED
CBLK = 1024
NBLK = BATCH // CBLK


def _tc_pipe_body(ft_ref, bias_ref, xt_hbm, out_ref, *scratch):
    bufs = scratch[:NBLK]
    sems = scratch[NBLK:]

    copies = []
    for j in range(NBLK):
        c = pltpu.make_async_copy(
            xt_hbm.at[:, pl.ds(j * CBLK, CBLK)], bufs[j], sems[j])
        c.start()
        copies.append(c)
    for j in range(NBLK):
        copies[j].wait()
        x = bufs[j][...]
        v0 = x[0:64, :]
        v1 = x[64:128, :]
        v2 = x[128:192, :]
        v3 = x[192:256, :]
        t = v0 * v1 + v2 * v3 + (v0 + v1) * (v2 + v3)
        inter = jnp.sum(t, axis=0, keepdims=True)
        ft = jnp.sum(ft_ref[:, pl.ds(j * CBLK, CBLK)], axis=0, keepdims=True)
        out_ref[:, pl.ds(j * CBLK, CBLK)] = bias_ref[0, 0] + ft + inter


def kernel(first_embeddings, second_embeddings, bias):
    xt = jnp.transpose(second_embeddings, (1, 2, 0)).reshape(ROWS, BATCH)
    ft = jnp.transpose(first_embeddings, (1, 0))
    xt = pltpu.with_memory_space_constraint(xt, pltpu.MemorySpace.HBM)
    out = pl.pallas_call(
        _tc_pipe_body,
        in_specs=[
            pl.BlockSpec((FIELDS, BATCH), lambda: (0, 0)),
            pl.BlockSpec((1, 1), lambda: (0, 0)),
            pl.BlockSpec(memory_space=pl.ANY),
        ],
        out_specs=pl.BlockSpec((1, BATCH), lambda: (0, 0)),
        out_shape=jax.ShapeDtypeStruct((1, BATCH), jnp.float32),
        scratch_shapes=(
            [pltpu.VMEM((ROWS, CBLK), jnp.float32) for _ in range(NBLK)]
            + [pltpu.SemaphoreType.DMA for _ in range(NBLK)]
        ),
    )(ft, bias.reshape(1, 1), xt)
    return out.reshape(BATCH)
